# baseline (speedup 1.0000x reference)
import functools

import jax
import jax.numpy as jnp
from jax import lax
from jax.experimental import pallas as pl
from jax.experimental.pallas import tpu as pltpu
from jax.experimental.pallas import tpu_sc as plsc

NC, NS, L = 2, 16, 16
NW = NC * NS

GROUP = 128
GROUPS_PER_CHUNK = 10
CHUNK = GROUP * GROUPS_PER_CHUNK


@functools.lru_cache(maxsize=None)
def _gather_fn(N, D, S):
    n_per_w = N // NW
    rows_w = n_per_w // GROUP
    n_chunks = rows_w // GROUPS_PER_CHUNK
    assert N == NW * n_per_w and n_per_w == rows_w * GROUP
    assert rows_w == n_chunks * GROUPS_PER_CHUNK
    mesh = plsc.VectorSubcoreMesh(core_axis_name="c", subcore_axis_name="s")

    @functools.partial(
        pl.kernel,
        mesh=mesh,
        compiler_params=pltpu.CompilerParams(use_tc_tiling_on_sc=False),
        out_type=jax.ShapeDtypeStruct((N, D), jnp.float32),
        scratch_types=[
            pltpu.VMEM((rows_w, GROUP), jnp.int32),
            pltpu.VMEM((CHUNK, D), jnp.float32),
            pltpu.VMEM((L,), jnp.int32),
            pltpu.SemaphoreType.DMA,
        ],
    )
    def gather_kernel(table_hbm, idx_hbm, seqlen_hbm, out_hbm,
                      idx_v, rows_v, seql_v, sem):
        wid = lax.axis_index("s") * NC + lax.axis_index("c")
        pltpu.sync_copy(idx_hbm.at[wid], idx_v)
        pltpu.sync_copy(seqlen_hbm, seql_v)
        seql = seql_v[...]
        lane = lax.iota(jnp.int32, L)
        elem0 = wid * n_per_w

        def fix_row(i, carry):
            for j in range(GROUP // L):
                v = idx_v[i, pl.ds(j * L, L)]
                col = lax.rem(elem0 + i * GROUP + j * L + lane, S)
                idx_v[i, pl.ds(j * L, L)] = jnp.where(
                    col < seql, v + 1, jnp.zeros_like(v))
            return carry

        lax.fori_loop(0, rows_w, fix_row, 0)

        base = wid * n_per_w
        for c in range(n_chunks):
            handles = []
            for g in range(GROUPS_PER_CHUNK):
                handles.append(pltpu.async_copy(
                    table_hbm.at[idx_v.at[c * GROUPS_PER_CHUNK + g]],
                    rows_v.at[pl.ds(g * GROUP, GROUP)], sem))
            for h in handles:
                h.wait()
            pltpu.sync_copy(rows_v, out_hbm.at[pl.ds(base + c * CHUNK, CHUNK)])

    return gather_kernel


def kernel(text, seq_len, text_embed_weight):
    B, S = text.shape
    N = B * S
    D = text_embed_weight.shape[1]
    idx2d = text.reshape(NW, N // (NW * GROUP), GROUP)
    seql_vec = jnp.full((L,), seq_len, dtype=jnp.int32)
    out = _gather_fn(N, D, S)(text_embed_weight, idx2d, seql_vec)
    return out.reshape(B, S, D)
